# baseline (speedup 1.0000x reference)
import functools

import jax
import jax.numpy as jnp
from jax import lax
from jax.experimental import pallas as pl
from jax.experimental.pallas import tpu as pltpu
from jax.experimental.pallas import tpu_sc as plsc

_BS, _P1, _C1 = 16, 4096, 128
_P2, _K = 1024, 32
_M1, _M2, _M3 = 128, 128, 256
_EPS = 1e-5
_NC, _NS = 2, 16
_NW = _NC * _NS
_TB = 256
_NR = _BS * _P2 * _K
_NQ = _BS * _P2
_TQ = 512


def _prep_body(xyz_ref, feat_ref, w1f_ref, w1x_ref, b1_ref, g_ref, tb_ref):
    xyz = xyz_ref[0]
    feat = feat_ref[0]
    p = lax.dot_general(xyz, w1x_ref[...], (((1,), (0,)), ((), ())),
                        preferred_element_type=jnp.float32)
    g = lax.dot_general(feat, w1f_ref[...], (((1,), (0,)), ((), ())),
                        preferred_element_type=jnp.float32)
    g_ref[0] = g + p + b1_ref[...]
    tb_ref[0] = jnp.concatenate(
        [p, xyz, jnp.zeros((_P1, _TB - _C1 - 3), jnp.float32)], axis=1)


def _prep(input_xyz, input_feature, w1f, w1x, b1):
    return pl.pallas_call(
        _prep_body,
        grid=(_BS,),
        in_specs=[
            pl.BlockSpec((1, _P1, 3), lambda b: (b, 0, 0)),
            pl.BlockSpec((1, _P1, _C1), lambda b: (b, 0, 0)),
            pl.BlockSpec((_C1, _M1), lambda b: (0, 0)),
            pl.BlockSpec((3, _M1), lambda b: (0, 0)),
            pl.BlockSpec((1, _M1), lambda b: (0, 0)),
        ],
        out_specs=[
            pl.BlockSpec((1, _P1, _M1), lambda b: (b, 0, 0)),
            pl.BlockSpec((1, _P1, _TB), lambda b: (b, 0, 0)),
        ],
        out_shape=[
            jax.ShapeDtypeStruct((_BS, _P1, _M1), jnp.float32),
            jax.ShapeDtypeStruct((_BS, _P1, _TB), jnp.float32),
        ],
    )(input_xyz, input_feature, w1f, w1x, b1)


def _sc_gather(table, idx, chunk):
    B = idx.shape[0]
    D = table.shape[1]
    rpw = B // _NW
    nchunk = rpw // chunk
    mesh = plsc.VectorSubcoreMesh(core_axis_name="c", subcore_axis_name="s",
                                  num_cores=_NC, num_subcores=_NS)

    @functools.partial(
        pl.kernel,
        out_type=jax.ShapeDtypeStruct((B, D), jnp.float32),
        mesh=mesh,
        scratch_types=[
            pltpu.VMEM((rpw,), jnp.int32),
            pltpu.VMEM((chunk, D), jnp.float32),
            pltpu.SemaphoreType.DMA,
        ],
    )
    def k(table_hbm, idx_hbm, out_hbm, idx_v, rows_v, sem):
        wid = lax.axis_index("s") * _NC + lax.axis_index("c")
        base = wid * rpw
        pltpu.sync_copy(idx_hbm.at[pl.ds(base, rpw)], idx_v)

        def body(c, carry):
            pltpu.async_copy(table_hbm.at[idx_v.at[pl.ds(c * chunk, chunk)]],
                             rows_v, sem).wait()
            pltpu.sync_copy(rows_v, out_hbm.at[pl.ds(base + c * chunk, chunk)])
            return carry

        lax.fori_loop(0, nchunk, body, 0)

    return k(table, idx)


def _topk_body(nxyz_ref, xyz_ref, nn_ref, d_ref):
    nx = nxyz_ref[0]
    x = xyz_ref[0]
    m = lax.dot_general(nx, x, (((1,), (1,)), ((), ())),
                        preferred_element_type=jnp.float32)
    ra = jnp.sum(nx * nx, axis=1, keepdims=True)
    rb = jnp.sum(x * x, axis=1)
    d_ref[...] = ra - 2.0 * m + rb[None, :]
    lane = lax.broadcasted_iota(jnp.int32, (_TQ, _P1), 1)
    col = lax.broadcasted_iota(jnp.int32, (_TQ, _K), 1)

    def body(i, nn):
        d = d_ref[...]
        v = jnp.min(d, axis=1, keepdims=True)
        idx = jnp.min(jnp.where(d == v, lane, _P1), axis=1).astype(jnp.int32)
        nn = jnp.where(col == i, idx[:, None], nn)
        d_ref[...] = jnp.where(lane == idx[:, None], jnp.inf, d)
        return nn

    nn_ref[0] = lax.fori_loop(0, _K, body,
                              jnp.zeros((_TQ, _K), jnp.int32), unroll=8)


def _topk(new_xyz, input_xyz):
    return pl.pallas_call(
        _topk_body,
        grid=(new_xyz.shape[0], _P2 // _TQ),
        in_specs=[
            pl.BlockSpec((1, _TQ, 3), lambda b, q: (b, q, 0)),
            pl.BlockSpec((1, _P1, 3), lambda b, q: (b, 0, 0)),
        ],
        out_specs=pl.BlockSpec((1, _TQ, _K), lambda b, q: (b, q, 0)),
        out_shape=jax.ShapeDtypeStruct((new_xyz.shape[0], _P2, _K), jnp.int32),
        scratch_shapes=[pltpu.VMEM((_TQ, _P1), jnp.float32)],
    )(new_xyz, input_xyz)


_SEGS = 4
_BSS = _BS // _SEGS
_NRS = _NR // _SEGS
_NQS = _NQ // _SEGS

_TR5 = 2048


def _stats_body(xg_ref, c_ref, o_ref):
    x = xg_ref[...].reshape(_TR5 // _K, _K, _M1) - c_ref[...][:, None, :]
    x = x.reshape(_TR5, _M1)
    s = jnp.sum(x, axis=0)
    q = jnp.sum(x * x, axis=0)

    @pl.when(pl.program_id(0) == 0)
    def _():
        o_ref[...] = jnp.zeros((8, _M1), jnp.float32)

    o_ref[...] += jnp.concatenate(
        [s[None], q[None], jnp.zeros((6, _M1), jnp.float32)], axis=0)


def _stats1(xg, cmat):
    return pl.pallas_call(
        _stats_body,
        grid=(_NRS // _TR5,),
        in_specs=[
            pl.BlockSpec((_TR5, _M1), lambda i: (i, 0)),
            pl.BlockSpec((_TR5 // _K, _M1), lambda i: (i, 0)),
        ],
        out_specs=pl.BlockSpec((8, _M1), lambda i: (0, 0)),
        out_shape=jax.ShapeDtypeStruct((8, _M1), jnp.float32),
    )(xg, cmat)


def _norm_coefs(st_ref, g_ref, be_ref, n):
    mean = jnp.sum(st_ref[:, 0:1, :], axis=0) / n
    var = jnp.maximum(jnp.sum(st_ref[:, 1:2, :], axis=0) / n - mean * mean, 0.0)
    a = g_ref[...] * lax.rsqrt(var + _EPS)
    c = be_ref[...] - mean * a
    return a, c


_TR6 = 1024


def _mlp2_body(xg_ref, c_ref, st_ref, g1_ref, be1_ref, w2_ref, b2_ref,
               x2_ref, o_ref):
    a, c = _norm_coefs(st_ref, g1_ref, be1_ref, jnp.float32(_NR))
    x1 = xg_ref[...].reshape(_TR6 // _K, _K, _M1) - c_ref[...][:, None, :]
    x1 = x1.reshape(_TR6, _M1)
    h1 = jnp.maximum(x1 * a + c, 0.0)
    x2 = lax.dot_general(h1, w2_ref[...], (((1,), (0,)), ((), ())),
                         preferred_element_type=jnp.float32) + b2_ref[...]
    x2_ref[...] = x2.astype(jnp.bfloat16)
    s = jnp.sum(x2, axis=0)
    q = jnp.sum(x2 * x2, axis=0)

    @pl.when(pl.program_id(0) == 0)
    def _():
        o_ref[...] = jnp.zeros((8, _M2), jnp.float32)

    o_ref[...] += jnp.concatenate(
        [s[None], q[None], jnp.zeros((6, _M2), jnp.float32)], axis=0)


def _mlp2(xg, cmat, st1, g1, be1, w2, b2):
    return pl.pallas_call(
        _mlp2_body,
        grid=(_NRS // _TR6,),
        in_specs=[
            pl.BlockSpec((_TR6, _M1), lambda i: (i, 0)),
            pl.BlockSpec((_TR6 // _K, _M1), lambda i: (i, 0)),
            pl.BlockSpec((_SEGS, 8, _M1), lambda i: (0, 0, 0)),
            pl.BlockSpec((1, _M1), lambda i: (0, 0)),
            pl.BlockSpec((1, _M1), lambda i: (0, 0)),
            pl.BlockSpec((_M1, _M2), lambda i: (0, 0)),
            pl.BlockSpec((1, _M2), lambda i: (0, 0)),
        ],
        out_specs=[
            pl.BlockSpec((_TR6, _M2), lambda i: (i, 0)),
            pl.BlockSpec((8, _M2), lambda i: (0, 0)),
        ],
        out_shape=[
            jax.ShapeDtypeStruct((_NRS, _M2), jnp.bfloat16),
            jax.ShapeDtypeStruct((8, _M2), jnp.float32),
        ],
    )(xg, cmat, st1, g1, be1, w2, b2)


_TR7 = 1024


def _mlp3_body(x2_ref, st_ref, g2_ref, be2_ref, w3_ref, b3_ref,
               mx_ref, mn_ref, o_ref):
    a, c = _norm_coefs(st_ref, g2_ref, be2_ref, jnp.float32(_NR))
    h2 = jnp.maximum(x2_ref[...].astype(jnp.float32) * a + c, 0.0)
    x3 = lax.dot_general(h2, w3_ref[...], (((1,), (0,)), ((), ())),
                         preferred_element_type=jnp.float32) + b3_ref[...]
    s = jnp.sum(x3, axis=0)
    q = jnp.sum(x3 * x3, axis=0)
    xr = x3.reshape(_TR7 // _K, _K, _M3)
    mx_ref[...] = jnp.max(xr, axis=1)
    mn_ref[...] = jnp.min(xr, axis=1)

    @pl.when(pl.program_id(0) == 0)
    def _():
        o_ref[...] = jnp.zeros((8, _M3), jnp.float32)

    o_ref[...] += jnp.concatenate(
        [s[None], q[None], jnp.zeros((6, _M3), jnp.float32)], axis=0)


def _mlp3(x2, st2, g2, be2, w3, b3):
    return pl.pallas_call(
        _mlp3_body,
        grid=(_NRS // _TR7,),
        in_specs=[
            pl.BlockSpec((_TR7, _M2), lambda i: (i, 0)),
            pl.BlockSpec((_SEGS, 8, _M2), lambda i: (0, 0, 0)),
            pl.BlockSpec((1, _M2), lambda i: (0, 0)),
            pl.BlockSpec((1, _M2), lambda i: (0, 0)),
            pl.BlockSpec((_M2, _M3), lambda i: (0, 0)),
            pl.BlockSpec((1, _M3), lambda i: (0, 0)),
        ],
        out_specs=[
            pl.BlockSpec((_TR7 // _K, _M3), lambda i: (i, 0)),
            pl.BlockSpec((_TR7 // _K, _M3), lambda i: (i, 0)),
            pl.BlockSpec((8, _M3), lambda i: (0, 0)),
        ],
        out_shape=[
            jax.ShapeDtypeStruct((_NQS, _M3), jnp.float32),
            jax.ShapeDtypeStruct((_NQS, _M3), jnp.float32),
            jax.ShapeDtypeStruct((8, _M3), jnp.float32),
        ],
    )(x2, st2, g2, be2, w3, b3)


_TP8 = 2048


def _final_body(mx_ref, mn_ref, st_ref, g3_ref, be3_ref, out_ref):
    a, c = _norm_coefs(st_ref, g3_ref, be3_ref, jnp.float32(_NR))
    v = jnp.where(a > 0.0, mx_ref[...], mn_ref[...])
    out_ref[...] = jnp.maximum(v * a + c, 0.0)


def _final(mx, mn, st3, g3, be3):
    return pl.pallas_call(
        _final_body,
        grid=(_NQS // _TP8,),
        in_specs=[
            pl.BlockSpec((_TP8, _M3), lambda i: (i, 0)),
            pl.BlockSpec((_TP8, _M3), lambda i: (i, 0)),
            pl.BlockSpec((_SEGS, 8, _M3), lambda i: (0, 0, 0)),
            pl.BlockSpec((1, _M3), lambda i: (0, 0)),
            pl.BlockSpec((1, _M3), lambda i: (0, 0)),
        ],
        out_specs=pl.BlockSpec((_TP8, _M3), lambda i: (i, 0)),
        out_shape=jax.ShapeDtypeStruct((_NQS, _M3), jnp.float32),
    )(mx, mn, st3, g3, be3)


def kernel(input_xyz, input_feature, idx_sample,
           W1, b1, g1, be1, W2, b2, g2, be2, W3, b3, g3, be3):
    w1f, w1x = W1[:_C1], W1[_C1:]
    g_tab, s_tab = _prep(input_xyz, input_feature, w1f, w1x,
                         b1.reshape(1, _M1))

    boff = (jnp.arange(_BS, dtype=jnp.int32) * _P1)[:, None]
    flat_s = (idx_sample + boff).reshape(-1)
    rows1 = _sc_gather(s_tab.reshape(_BS * _P1, _TB), flat_s, 128)
    cmat = rows1[:, :_C1]
    new_xyz = rows1[:, _C1:_C1 + 3].reshape(_BS, _P2, 3)
    gflat = g_tab.reshape(_BS * _P1, _M1)

    nns, xgs, cms = [], [], []
    for s in range(_SEGS):
        b0 = s * _BSS
        nn_s = _topk(lax.slice_in_dim(new_xyz, b0, b0 + _BSS, axis=0),
                     lax.slice_in_dim(input_xyz, b0, b0 + _BSS, axis=0))
        fn = (nn_s + boff[b0:b0 + _BSS][:, :, None]).reshape(-1)
        nns.append(nn_s)
        xgs.append(_sc_gather(gflat, fn, 128))
        cms.append(lax.slice_in_dim(cmat, s * _NQS, (s + 1) * _NQS, axis=0))

    st1 = jnp.stack([_stats1(xgs[s], cms[s]) for s in range(_SEGS)])
    g1r, be1r = g1.reshape(1, _M1), be1.reshape(1, _M1)
    x2s, st2s = [], []
    for s in range(_SEGS):
        x2, st2 = _mlp2(xgs[s], cms[s], st1, g1r, be1r, W2, b2.reshape(1, _M2))
        x2s.append(x2)
        st2s.append(st2)
    st2 = jnp.stack(st2s)
    g2r, be2r = g2.reshape(1, _M2), be2.reshape(1, _M2)
    m3 = [_mlp3(x2s[s], st2, g2r, be2r, W3, b3.reshape(1, _M3))
          for s in range(_SEGS)]
    st3 = jnp.stack([t[2] for t in m3])
    g3r, be3r = g3.reshape(1, _M3), be3.reshape(1, _M3)
    nf = jnp.concatenate([_final(m3[s][0], m3[s][1], st3, g3r, be3r)
                          for s in range(_SEGS)], axis=0)
    return (jnp.concatenate(nns, axis=0), new_xyz,
            nf.reshape(_BS, _P2, _M3))


# speedup vs baseline: 1.1121x; 1.1121x over previous
import functools

import jax
import jax.numpy as jnp
from jax import lax
from jax.experimental import pallas as pl
from jax.experimental.pallas import tpu as pltpu
from jax.experimental.pallas import tpu_sc as plsc

_BS, _P1, _C1 = 16, 4096, 128
_P2, _K = 1024, 32
_M1, _M2, _M3 = 128, 128, 256
_EPS = 1e-5
_NC, _NS = 2, 16
_NW = _NC * _NS
_TB = 256
_NR = _BS * _P2 * _K
_NQ = _BS * _P2
_TQ = 512


def _prep_body(xyz_ref, feat_ref, w1f_ref, w1x_ref, b1_ref, g_ref, tb_ref):
    xyz = xyz_ref[0]
    feat = feat_ref[0]
    p = lax.dot_general(xyz, w1x_ref[...], (((1,), (0,)), ((), ())),
                        preferred_element_type=jnp.float32)
    g = lax.dot_general(feat, w1f_ref[...], (((1,), (0,)), ((), ())),
                        preferred_element_type=jnp.float32)
    g_ref[0] = g + p + b1_ref[...]
    tb_ref[0] = jnp.concatenate(
        [p, xyz, jnp.zeros((_P1, _TB - _C1 - 3), jnp.float32)], axis=1)


def _prep(input_xyz, input_feature, w1f, w1x, b1):
    return pl.pallas_call(
        _prep_body,
        grid=(_BS,),
        in_specs=[
            pl.BlockSpec((1, _P1, 3), lambda b: (b, 0, 0)),
            pl.BlockSpec((1, _P1, _C1), lambda b: (b, 0, 0)),
            pl.BlockSpec((_C1, _M1), lambda b: (0, 0)),
            pl.BlockSpec((3, _M1), lambda b: (0, 0)),
            pl.BlockSpec((1, _M1), lambda b: (0, 0)),
        ],
        out_specs=[
            pl.BlockSpec((1, _P1, _M1), lambda b: (b, 0, 0)),
            pl.BlockSpec((1, _P1, _TB), lambda b: (b, 0, 0)),
        ],
        out_shape=[
            jax.ShapeDtypeStruct((_BS, _P1, _M1), jnp.float32),
            jax.ShapeDtypeStruct((_BS, _P1, _TB), jnp.float32),
        ],
    )(input_xyz, input_feature, w1f, w1x, b1)


def _sc_gather(table, idx, chunk):
    B = idx.shape[0]
    D = table.shape[1]
    rpw = B // _NW
    nchunk = rpw // chunk
    mesh = plsc.VectorSubcoreMesh(core_axis_name="c", subcore_axis_name="s",
                                  num_cores=_NC, num_subcores=_NS)

    @functools.partial(
        pl.kernel,
        out_type=jax.ShapeDtypeStruct((B, D), jnp.float32),
        mesh=mesh,
        scratch_types=[
            pltpu.VMEM((rpw,), jnp.int32),
            pltpu.VMEM((chunk, D), jnp.float32),
            pltpu.SemaphoreType.DMA,
        ],
    )
    def k(table_hbm, idx_hbm, out_hbm, idx_v, rows_v, sem):
        wid = lax.axis_index("s") * _NC + lax.axis_index("c")
        base = wid * rpw
        pltpu.sync_copy(idx_hbm.at[pl.ds(base, rpw)], idx_v)

        def body(c, carry):
            pltpu.async_copy(table_hbm.at[idx_v.at[pl.ds(c * chunk, chunk)]],
                             rows_v, sem).wait()
            pltpu.sync_copy(rows_v, out_hbm.at[pl.ds(base + c * chunk, chunk)])
            return carry

        lax.fori_loop(0, nchunk, body, 0)

    return k(table, idx)


def _topk_body(nxyz_ref, xyz_ref, nn_ref, d_ref):
    nx = nxyz_ref[0]
    x = xyz_ref[0]
    m = lax.dot_general(nx, x, (((1,), (1,)), ((), ())),
                        preferred_element_type=jnp.float32)
    ra = jnp.sum(nx * nx, axis=1, keepdims=True)
    rb = jnp.sum(x * x, axis=1)
    d_ref[...] = ra - 2.0 * m + rb[None, :]
    lane = lax.broadcasted_iota(jnp.int32, (_TQ, _P1), 1).astype(jnp.float32)
    col = lax.broadcasted_iota(jnp.int32, (_TQ, _K), 1)

    def body(i, nn):
        d = d_ref[...]
        v = jnp.min(d, axis=1, keepdims=True)
        idx = jnp.min(jnp.where(d == v, lane, jnp.float32(_P1)), axis=1,
                      keepdims=True)
        nn = jnp.where(col == i, idx.astype(jnp.int32), nn)
        d_ref[...] = jnp.where(lane == idx, jnp.inf, d)
        return nn

    nn_ref[0] = lax.fori_loop(0, _K, body,
                              jnp.zeros((_TQ, _K), jnp.int32), unroll=4)


def _topk(new_xyz, input_xyz):
    return pl.pallas_call(
        _topk_body,
        grid=(new_xyz.shape[0], _P2 // _TQ),
        in_specs=[
            pl.BlockSpec((1, _TQ, 3), lambda b, q: (b, q, 0)),
            pl.BlockSpec((1, _P1, 3), lambda b, q: (b, 0, 0)),
        ],
        out_specs=pl.BlockSpec((1, _TQ, _K), lambda b, q: (b, q, 0)),
        out_shape=jax.ShapeDtypeStruct((new_xyz.shape[0], _P2, _K), jnp.int32),
        scratch_shapes=[pltpu.VMEM((_TQ, _P1), jnp.float32)],
    )(new_xyz, input_xyz)


_SEGS = 4
_BSS = _BS // _SEGS
_NRS = _NR // _SEGS
_NQS = _NQ // _SEGS

_TR5 = 2048


def _stats_body(xg_ref, c_ref, o_ref):
    x = xg_ref[...].reshape(_TR5 // _K, _K, _M1) - c_ref[...][:, None, :]
    x = x.reshape(_TR5, _M1)
    s = jnp.sum(x, axis=0)
    q = jnp.sum(x * x, axis=0)

    @pl.when(pl.program_id(0) == 0)
    def _():
        o_ref[...] = jnp.zeros((8, _M1), jnp.float32)

    o_ref[...] += jnp.concatenate(
        [s[None], q[None], jnp.zeros((6, _M1), jnp.float32)], axis=0)


def _stats1(xg, cmat):
    return pl.pallas_call(
        _stats_body,
        grid=(_NRS // _TR5,),
        in_specs=[
            pl.BlockSpec((_TR5, _M1), lambda i: (i, 0)),
            pl.BlockSpec((_TR5 // _K, _M1), lambda i: (i, 0)),
        ],
        out_specs=pl.BlockSpec((8, _M1), lambda i: (0, 0)),
        out_shape=jax.ShapeDtypeStruct((8, _M1), jnp.float32),
    )(xg, cmat)


def _norm_coefs(st_ref, g_ref, be_ref, n):
    mean = jnp.sum(st_ref[:, 0:1, :], axis=0) / n
    var = jnp.maximum(jnp.sum(st_ref[:, 1:2, :], axis=0) / n - mean * mean, 0.0)
    a = g_ref[...] * lax.rsqrt(var + _EPS)
    c = be_ref[...] - mean * a
    return a, c


_TR6 = 1024


def _mlp2_body(xg_ref, c_ref, st_ref, g1_ref, be1_ref, w2_ref, b2_ref,
               x2_ref, o_ref):
    a, c = _norm_coefs(st_ref, g1_ref, be1_ref, jnp.float32(_NR))
    x1 = xg_ref[...].reshape(_TR6 // _K, _K, _M1) - c_ref[...][:, None, :]
    x1 = x1.reshape(_TR6, _M1)
    h1 = jnp.maximum(x1 * a + c, 0.0)
    x2 = lax.dot_general(h1, w2_ref[...], (((1,), (0,)), ((), ())),
                         preferred_element_type=jnp.float32) + b2_ref[...]
    x2_ref[...] = x2.astype(jnp.bfloat16)
    s = jnp.sum(x2, axis=0)
    q = jnp.sum(x2 * x2, axis=0)

    @pl.when(pl.program_id(0) == 0)
    def _():
        o_ref[...] = jnp.zeros((8, _M2), jnp.float32)

    o_ref[...] += jnp.concatenate(
        [s[None], q[None], jnp.zeros((6, _M2), jnp.float32)], axis=0)


def _mlp2(xg, cmat, st1, g1, be1, w2, b2):
    return pl.pallas_call(
        _mlp2_body,
        grid=(_NRS // _TR6,),
        in_specs=[
            pl.BlockSpec((_TR6, _M1), lambda i: (i, 0)),
            pl.BlockSpec((_TR6 // _K, _M1), lambda i: (i, 0)),
            pl.BlockSpec((_SEGS, 8, _M1), lambda i: (0, 0, 0)),
            pl.BlockSpec((1, _M1), lambda i: (0, 0)),
            pl.BlockSpec((1, _M1), lambda i: (0, 0)),
            pl.BlockSpec((_M1, _M2), lambda i: (0, 0)),
            pl.BlockSpec((1, _M2), lambda i: (0, 0)),
        ],
        out_specs=[
            pl.BlockSpec((_TR6, _M2), lambda i: (i, 0)),
            pl.BlockSpec((8, _M2), lambda i: (0, 0)),
        ],
        out_shape=[
            jax.ShapeDtypeStruct((_NRS, _M2), jnp.bfloat16),
            jax.ShapeDtypeStruct((8, _M2), jnp.float32),
        ],
    )(xg, cmat, st1, g1, be1, w2, b2)


_TR7 = 1024


def _mlp3_body(x2_ref, st_ref, g2_ref, be2_ref, w3_ref, b3_ref,
               mx_ref, mn_ref, o_ref):
    a, c = _norm_coefs(st_ref, g2_ref, be2_ref, jnp.float32(_NR))
    h2 = jnp.maximum(x2_ref[...].astype(jnp.float32) * a + c, 0.0)
    x3 = lax.dot_general(h2, w3_ref[...], (((1,), (0,)), ((), ())),
                         preferred_element_type=jnp.float32) + b3_ref[...]
    s = jnp.sum(x3, axis=0)
    q = jnp.sum(x3 * x3, axis=0)
    xr = x3.reshape(_TR7 // _K, _K, _M3)
    mx_ref[...] = jnp.max(xr, axis=1)
    mn_ref[...] = jnp.min(xr, axis=1)

    @pl.when(pl.program_id(0) == 0)
    def _():
        o_ref[...] = jnp.zeros((8, _M3), jnp.float32)

    o_ref[...] += jnp.concatenate(
        [s[None], q[None], jnp.zeros((6, _M3), jnp.float32)], axis=0)


def _mlp3(x2, st2, g2, be2, w3, b3):
    return pl.pallas_call(
        _mlp3_body,
        grid=(_NRS // _TR7,),
        in_specs=[
            pl.BlockSpec((_TR7, _M2), lambda i: (i, 0)),
            pl.BlockSpec((_SEGS, 8, _M2), lambda i: (0, 0, 0)),
            pl.BlockSpec((1, _M2), lambda i: (0, 0)),
            pl.BlockSpec((1, _M2), lambda i: (0, 0)),
            pl.BlockSpec((_M2, _M3), lambda i: (0, 0)),
            pl.BlockSpec((1, _M3), lambda i: (0, 0)),
        ],
        out_specs=[
            pl.BlockSpec((_TR7 // _K, _M3), lambda i: (i, 0)),
            pl.BlockSpec((_TR7 // _K, _M3), lambda i: (i, 0)),
            pl.BlockSpec((8, _M3), lambda i: (0, 0)),
        ],
        out_shape=[
            jax.ShapeDtypeStruct((_NQS, _M3), jnp.float32),
            jax.ShapeDtypeStruct((_NQS, _M3), jnp.float32),
            jax.ShapeDtypeStruct((8, _M3), jnp.float32),
        ],
    )(x2, st2, g2, be2, w3, b3)


_TP8 = 2048


def _final_body(mx_ref, mn_ref, st_ref, g3_ref, be3_ref, out_ref):
    a, c = _norm_coefs(st_ref, g3_ref, be3_ref, jnp.float32(_NR))
    v = jnp.where(a > 0.0, mx_ref[...], mn_ref[...])
    out_ref[...] = jnp.maximum(v * a + c, 0.0)


def _final(mx, mn, st3, g3, be3):
    return pl.pallas_call(
        _final_body,
        grid=(_NQS // _TP8,),
        in_specs=[
            pl.BlockSpec((_TP8, _M3), lambda i: (i, 0)),
            pl.BlockSpec((_TP8, _M3), lambda i: (i, 0)),
            pl.BlockSpec((_SEGS, 8, _M3), lambda i: (0, 0, 0)),
            pl.BlockSpec((1, _M3), lambda i: (0, 0)),
            pl.BlockSpec((1, _M3), lambda i: (0, 0)),
        ],
        out_specs=pl.BlockSpec((_TP8, _M3), lambda i: (i, 0)),
        out_shape=jax.ShapeDtypeStruct((_NQS, _M3), jnp.float32),
    )(mx, mn, st3, g3, be3)


def kernel(input_xyz, input_feature, idx_sample,
           W1, b1, g1, be1, W2, b2, g2, be2, W3, b3, g3, be3):
    w1f, w1x = W1[:_C1], W1[_C1:]
    g_tab, s_tab = _prep(input_xyz, input_feature, w1f, w1x,
                         b1.reshape(1, _M1))

    boff = (jnp.arange(_BS, dtype=jnp.int32) * _P1)[:, None]
    flat_s = (idx_sample + boff).reshape(-1)
    rows1 = _sc_gather(s_tab.reshape(_BS * _P1, _TB), flat_s, 128)
    cmat = rows1[:, :_C1]
    new_xyz = rows1[:, _C1:_C1 + 3].reshape(_BS, _P2, 3)
    gflat = g_tab.reshape(_BS * _P1, _M1)

    nns, xgs, cms = [], [], []
    for s in range(_SEGS):
        b0 = s * _BSS
        nn_s = _topk(lax.slice_in_dim(new_xyz, b0, b0 + _BSS, axis=0),
                     lax.slice_in_dim(input_xyz, b0, b0 + _BSS, axis=0))
        fn = (nn_s + boff[b0:b0 + _BSS][:, :, None]).reshape(-1)
        nns.append(nn_s)
        xgs.append(_sc_gather(gflat, fn, 128))
        cms.append(lax.slice_in_dim(cmat, s * _NQS, (s + 1) * _NQS, axis=0))

    st1 = jnp.stack([_stats1(xgs[s], cms[s]) for s in range(_SEGS)])
    g1r, be1r = g1.reshape(1, _M1), be1.reshape(1, _M1)
    x2s, st2s = [], []
    for s in range(_SEGS):
        x2, st2 = _mlp2(xgs[s], cms[s], st1, g1r, be1r, W2, b2.reshape(1, _M2))
        x2s.append(x2)
        st2s.append(st2)
    st2 = jnp.stack(st2s)
    g2r, be2r = g2.reshape(1, _M2), be2.reshape(1, _M2)
    m3 = [_mlp3(x2s[s], st2, g2r, be2r, W3, b3.reshape(1, _M3))
          for s in range(_SEGS)]
    st3 = jnp.stack([t[2] for t in m3])
    g3r, be3r = g3.reshape(1, _M3), be3.reshape(1, _M3)
    nf = jnp.concatenate([_final(m3[s][0], m3[s][1], st3, g3r, be3r)
                          for s in range(_SEGS)], axis=0)
    return (jnp.concatenate(nns, axis=0), new_xyz,
            nf.reshape(_BS, _P2, _M3))
